# Initial kernel scaffold; baseline (speedup 1.0000x reference)
import jax
import jax.numpy as jnp
from jax.experimental import pallas as pl


def kernel(given4rank1_stimulus_set, kernel_gate_weights, embed_table, w0, w1):
    raise NotImplementedError("write your pallas kernel here")


# speedup vs baseline: 9.1500x; 9.1500x over previous
import functools

import jax
import jax.numpy as jnp
from jax import lax
from jax.experimental import pallas as pl
from jax.experimental.pallas import tpu as pltpu
from jax.experimental.pallas import tpu_sc as plsc

B = 16384
N_STIMULI = 30
NPAD = 32
RHO = 2.0
BETA = 10.0

NUM_CORES = 2
NUM_SUBCORES = 16
NW = NUM_CORES * NUM_SUBCORES
ROWS_PER_W = B // NW
LANES = 16
CHUNKS = ROWS_PER_W // LANES


N_DIM = 10


def _table_body(et_ref, ett_ref, w0_ref, w1_ref, out_ref):
    d2_0 = jnp.zeros((NPAD, NPAD), jnp.float32)
    d2_1 = jnp.zeros((NPAD, NPAD), jnp.float32)
    for d in range(N_DIM):
        diff = et_ref[:, d:d + 1] - ett_ref[d:d + 1, :]
        sq = diff * diff
        d2_0 = d2_0 + w0_ref[d:d + 1, :] * sq
        d2_1 = d2_1 + w1_ref[d:d + 1, :] * sq
    out_ref[0, :, :] = jnp.exp(-BETA * jnp.sqrt(d2_0))
    out_ref[1, :, :] = jnp.exp(-BETA * jnp.sqrt(d2_1))


def _make_tables(et32, w0, w1):
    return pl.pallas_call(
        _table_body,
        out_shape=jax.ShapeDtypeStruct((2, NPAD, NPAD), jnp.float32),
    )(et32, et32.T, w0.reshape(N_DIM, 1), w1.reshape(N_DIM, 1))


def _sc_body(stim_hbm, gates_hbm, tab_hbm, out_hbm,
             stim_v, gates_v, tab_v, out_v):
    wid = lax.axis_index("s") * NUM_CORES + lax.axis_index("c")
    base = wid * ROWS_PER_W
    pltpu.sync_copy(stim_hbm.at[pl.ds(base * 5, ROWS_PER_W * 5)], stim_v)
    pltpu.sync_copy(gates_hbm.at[pl.ds(base * 2, ROWS_PER_W * 2)], gates_v)
    pltpu.sync_copy(tab_hbm, tab_v)

    iota = lax.iota(jnp.int32, LANES)

    def chunk(c, carry):
        rows = c * LANES + iota
        r5 = rows * 5
        r2 = rows * 2
        q = plsc.load_gather(stim_v, [r5])
        g0 = plsc.load_gather(gates_v, [r2])
        g1 = plsc.load_gather(gates_v, [r2 + 1])
        qt = q * NPAD
        s = []
        for k in range(4):
            r = plsc.load_gather(stim_v, [r5 + (k + 1)])
            p = qt + r
            s0 = plsc.load_gather(tab_v, [p])
            s1 = plsc.load_gather(tab_v, [p + NPAD * NPAD])
            s.append(g0 * s0 + g1 * s1)
        inv = 1.0 / ((s[0] + s[1]) + (s[2] + s[3]))
        r4 = rows * 4
        for k in range(4):
            plsc.store_scatter(out_v, [r4 + k], s[k] * inv)
        return carry

    lax.fori_loop(0, CHUNKS, chunk, 0)
    pltpu.sync_copy(out_v, out_hbm.at[pl.ds(base * 4, ROWS_PER_W * 4)])


@functools.lru_cache(maxsize=1)
def _sc_rank():
    return pl.kernel(
        _sc_body,
        out_type=jax.ShapeDtypeStruct((B * 4,), jnp.float32),
        mesh=plsc.VectorSubcoreMesh(core_axis_name="c", subcore_axis_name="s"),
        compiler_params=pltpu.CompilerParams(needs_layout_passes=False),
        scratch_types=[
            pltpu.VMEM((ROWS_PER_W * 5,), jnp.int32),
            pltpu.VMEM((ROWS_PER_W * 2,), jnp.float32),
            pltpu.VMEM((2 * NPAD * NPAD,), jnp.float32),
            pltpu.VMEM((ROWS_PER_W * 4,), jnp.float32),
        ],
    )


def kernel(given4rank1_stimulus_set, kernel_gate_weights, embed_table, w0, w1):
    et32 = jnp.zeros((NPAD, embed_table.shape[1]), jnp.float32)
    et32 = et32.at[: embed_table.shape[0]].set(embed_table)
    tables = _make_tables(et32, w0, w1)
    flat = _sc_rank()(
        given4rank1_stimulus_set.astype(jnp.int32).reshape(-1),
        kernel_gate_weights.reshape(-1),
        tables.reshape(-1),
    )
    return flat.reshape(B, 4)


# speedup vs baseline: 19.7487x; 2.1583x over previous
import functools

import jax
import jax.numpy as jnp
from jax import lax
from jax.experimental import pallas as pl
from jax.experimental.pallas import tpu as pltpu
from jax.experimental.pallas import tpu_sc as plsc

B = 16384
N_STIMULI = 30
N_DIM = 10
NPAD = 32
BETA = 10.0

NUM_CORES = 2
NUM_SUBCORES = 16
NW = NUM_CORES * NUM_SUBCORES
RPW = B // NW
LANES = 16
CHUNKS = RPW // LANES


def _table_body(et_ref, ett_ref, w0_ref, w1_ref, out_ref):
    d2_0 = jnp.zeros((NPAD, NPAD), jnp.float32)
    d2_1 = jnp.zeros((NPAD, NPAD), jnp.float32)
    for d in range(N_DIM):
        diff = et_ref[:, d:d + 1] - ett_ref[d:d + 1, :]
        sq = diff * diff
        d2_0 = d2_0 + w0_ref[0:1, d:d + 1] * sq
        d2_1 = d2_1 + w1_ref[0:1, d:d + 1] * sq
    out_ref[0, :, :] = jnp.exp(-BETA * jnp.sqrt(d2_0))
    out_ref[1, :, :] = jnp.exp(-BETA * jnp.sqrt(d2_1))


def _make_tables(et32, w0, w1):
    return pl.pallas_call(
        _table_body,
        out_shape=jax.ShapeDtypeStruct((2, NPAD, NPAD), jnp.float32),
    )(et32, et32.T, w0.reshape(1, N_DIM), w1.reshape(1, N_DIM))


def _sc_body(stim_hbm, gates_hbm, tab_hbm, out_hbm,
             q_v, r_v, g_v, tab_v, out_v):
    wid = lax.axis_index("s") * NUM_CORES + lax.axis_index("c")
    base = wid * RPW
    pltpu.sync_copy(stim_hbm.at[pl.ds(base, RPW)], q_v)
    for k in range(4):
        pltpu.sync_copy(stim_hbm.at[pl.ds((k + 1) * B + base, RPW)],
                        r_v.at[pl.ds(k * RPW, RPW)])
    pltpu.sync_copy(gates_hbm.at[pl.ds(base, RPW)], g_v.at[pl.ds(0, RPW)])
    pltpu.sync_copy(gates_hbm.at[pl.ds(B + base, RPW)], g_v.at[pl.ds(RPW, RPW)])
    pltpu.sync_copy(tab_hbm, tab_v)

    def chunk(c, carry):
        o = c * LANES
        q = q_v[pl.ds(o, LANES)]
        g0 = g_v[pl.ds(o, LANES)]
        g1 = g_v[pl.ds(RPW + o, LANES)]
        qt = q * NPAD
        s = []
        for k in range(4):
            r = r_v[pl.ds(k * RPW + o, LANES)]
            p = qt + r
            s0 = plsc.load_gather(tab_v, [p])
            s1 = plsc.load_gather(tab_v, [p + NPAD * NPAD])
            s.append(g0 * s0 + g1 * s1)
        inv = 1.0 / ((s[0] + s[1]) + (s[2] + s[3]))
        for k in range(4):
            out_v[pl.ds(k * RPW + o, LANES)] = s[k] * inv
        return carry

    lax.fori_loop(0, CHUNKS, chunk, 0)
    for k in range(4):
        pltpu.sync_copy(out_v.at[pl.ds(k * RPW, RPW)],
                        out_hbm.at[pl.ds(k * B + base, RPW)])


@functools.lru_cache(maxsize=1)
def _sc_rank():
    return pl.kernel(
        _sc_body,
        out_type=jax.ShapeDtypeStruct((4 * B,), jnp.float32),
        mesh=plsc.VectorSubcoreMesh(core_axis_name="c", subcore_axis_name="s"),
        compiler_params=pltpu.CompilerParams(needs_layout_passes=False),
        scratch_types=[
            pltpu.VMEM((RPW,), jnp.int32),
            pltpu.VMEM((4 * RPW,), jnp.int32),
            pltpu.VMEM((2 * RPW,), jnp.float32),
            pltpu.VMEM((2 * NPAD * NPAD,), jnp.float32),
            pltpu.VMEM((4 * RPW,), jnp.float32),
        ],
    )


def kernel(given4rank1_stimulus_set, kernel_gate_weights, embed_table, w0, w1):
    et32 = jnp.zeros((NPAD, N_DIM), jnp.float32)
    et32 = et32.at[: embed_table.shape[0]].set(embed_table)
    tables = _make_tables(et32, w0, w1)
    flat = _sc_rank()(
        given4rank1_stimulus_set.astype(jnp.int32).T.reshape(-1),
        kernel_gate_weights.T.reshape(-1),
        tables.reshape(-1),
    )
    return flat.reshape(4, B).T


# speedup vs baseline: 21.7859x; 1.1032x over previous
import functools

import jax
import jax.numpy as jnp
from jax import lax
from jax.experimental import pallas as pl
from jax.experimental.pallas import tpu as pltpu
from jax.experimental.pallas import tpu_sc as plsc

B = 16384
N_STIMULI = 30
N_DIM = 10
NPAD = 32
BETA = 10.0

NUM_CORES = 2
NUM_SUBCORES = 16
NW = NUM_CORES * NUM_SUBCORES
RPW = B // NW
LANES = 16
CHUNKS = RPW // LANES


def _table_body(et_ref, ett_ref, w0_ref, w1_ref, out_ref):
    d2_0 = jnp.zeros((NPAD, NPAD), jnp.float32)
    d2_1 = jnp.zeros((NPAD, NPAD), jnp.float32)
    for d in range(N_DIM):
        diff = et_ref[:, d:d + 1] - ett_ref[d:d + 1, :]
        sq = diff * diff
        d2_0 = d2_0 + w0_ref[0:1, d:d + 1] * sq
        d2_1 = d2_1 + w1_ref[0:1, d:d + 1] * sq
    out_ref[0, :, :] = jnp.exp(-BETA * jnp.sqrt(d2_0))
    out_ref[1, :, :] = jnp.exp(-BETA * jnp.sqrt(d2_1))


def _make_tables(et32, w0, w1):
    return pl.pallas_call(
        _table_body,
        out_shape=jax.ShapeDtypeStruct((2, NPAD, NPAD), jnp.float32),
    )(et32, et32.T, w0.reshape(1, N_DIM), w1.reshape(1, N_DIM))


def _sc_body(stim_hbm, gates_hbm, tab_hbm, out_hbm,
             q_v, r_v, g_v, tab_v, out_v, sem):
    wid = lax.axis_index("s") * NUM_CORES + lax.axis_index("c")
    base = wid * RPW
    copies = [pltpu.async_copy(stim_hbm.at[pl.ds(base, RPW)], q_v, sem)]
    for k in range(4):
        copies.append(pltpu.async_copy(
            stim_hbm.at[pl.ds((k + 1) * B + base, RPW)],
            r_v.at[pl.ds(k * RPW, RPW)], sem))
    copies.append(pltpu.async_copy(gates_hbm.at[pl.ds(base, RPW)],
                                   g_v.at[pl.ds(0, RPW)], sem))
    copies.append(pltpu.async_copy(gates_hbm.at[pl.ds(B + base, RPW)],
                                   g_v.at[pl.ds(RPW, RPW)], sem))
    copies.append(pltpu.async_copy(tab_hbm, tab_v, sem))
    for c in copies:
        c.wait()

    def chunk(c, carry):
        o = c * LANES
        q = q_v[pl.ds(o, LANES)]
        g0 = g_v[pl.ds(o, LANES)]
        g1 = g_v[pl.ds(RPW + o, LANES)]
        qt = q * NPAD
        s = []
        for k in range(4):
            r = r_v[pl.ds(k * RPW + o, LANES)]
            p = qt + r
            s0 = plsc.load_gather(tab_v, [p])
            s1 = plsc.load_gather(tab_v, [p + NPAD * NPAD])
            s.append(g0 * s0 + g1 * s1)
        inv = 1.0 / ((s[0] + s[1]) + (s[2] + s[3]))
        for k in range(4):
            out_v[pl.ds(k * RPW + o, LANES)] = s[k] * inv
        return carry

    lax.fori_loop(0, CHUNKS, chunk, 0)
    outs = [pltpu.async_copy(out_v.at[pl.ds(k * RPW, RPW)],
                             out_hbm.at[pl.ds(k * B + base, RPW)], sem)
            for k in range(4)]
    for c in outs:
        c.wait()


@functools.lru_cache(maxsize=1)
def _sc_rank():
    return pl.kernel(
        _sc_body,
        out_type=jax.ShapeDtypeStruct((4 * B,), jnp.float32),
        mesh=plsc.VectorSubcoreMesh(core_axis_name="c", subcore_axis_name="s"),
        compiler_params=pltpu.CompilerParams(needs_layout_passes=False),
        scratch_types=[
            pltpu.VMEM((RPW,), jnp.int32),
            pltpu.VMEM((4 * RPW,), jnp.int32),
            pltpu.VMEM((2 * RPW,), jnp.float32),
            pltpu.VMEM((2 * NPAD * NPAD,), jnp.float32),
            pltpu.VMEM((4 * RPW,), jnp.float32),
            pltpu.SemaphoreType.DMA,
        ],
    )


def kernel(given4rank1_stimulus_set, kernel_gate_weights, embed_table, w0, w1):
    et32 = jnp.zeros((NPAD, N_DIM), jnp.float32)
    et32 = et32.at[: embed_table.shape[0]].set(embed_table)
    tables = _make_tables(et32, w0, w1)
    flat = _sc_rank()(
        given4rank1_stimulus_set.astype(jnp.int32).T.reshape(-1),
        kernel_gate_weights.T.reshape(-1),
        tables.reshape(-1),
    )
    return flat.reshape(4, B).T


# speedup vs baseline: 23.6360x; 1.0849x over previous
import functools

import jax
import jax.numpy as jnp
from jax import lax
from jax.experimental import pallas as pl
from jax.experimental.pallas import tpu as pltpu
from jax.experimental.pallas import tpu_sc as plsc

B = 16384
N_STIMULI = 30
N_DIM = 10
NPAD = 32
BETA = 10.0

NUM_CORES = 2
NUM_SUBCORES = 16
NW = NUM_CORES * NUM_SUBCORES
RPW = B // NW
LANES = 16
CHUNKS = RPW // LANES


def _table_body(ett_ref, w0_ref, w1_ref, out_ref):
    ett = jnp.concatenate(
        [ett_ref[...], jnp.zeros((N_DIM, 1), jnp.float32)], axis=1)
    et = ett.T
    d2_0 = jnp.zeros((NPAD, NPAD), jnp.float32)
    d2_1 = jnp.zeros((NPAD, NPAD), jnp.float32)
    for d in range(N_DIM):
        diff = et[:, d:d + 1] - ett[d:d + 1, :]
        sq = diff * diff
        d2_0 = d2_0 + w0_ref[0:1, d:d + 1] * sq
        d2_1 = d2_1 + w1_ref[0:1, d:d + 1] * sq
    for t, d2 in enumerate((d2_0, d2_1)):
        s = jnp.exp(-BETA * jnp.sqrt(d2))
        for qb in range(4):
            row = jnp.concatenate(
                [s[8 * qb + j: 8 * qb + j + 1, :] for j in range(8)], axis=1)
            out_ref[pl.ds(t * 1024 + qb * 256, 256)] = row.reshape(256)


def _make_tables(embed_table, w0, w1):
    return pl.pallas_call(
        _table_body,
        out_shape=jax.ShapeDtypeStruct((2 * NPAD * NPAD,), jnp.float32),
    )(embed_table.T, w0.reshape(1, N_DIM), w1.reshape(1, N_DIM))


def _sc_body(stim_hbm, gates_hbm, tab_hbm, out_hbm,
             q_v, r_v, g_v, tab_v, out_v, sem):
    wid = lax.axis_index("s") * NUM_CORES + lax.axis_index("c")
    base = wid * RPW
    copies = [pltpu.async_copy(stim_hbm.at[pl.ds(base, RPW)], q_v, sem)]
    for k in range(4):
        copies.append(pltpu.async_copy(
            stim_hbm.at[pl.ds((k + 1) * B + base, RPW)],
            r_v.at[pl.ds(k * RPW, RPW)], sem))
    copies.append(pltpu.async_copy(gates_hbm.at[pl.ds(base, RPW)],
                                   g_v.at[pl.ds(0, RPW)], sem))
    copies.append(pltpu.async_copy(gates_hbm.at[pl.ds(B + base, RPW)],
                                   g_v.at[pl.ds(RPW, RPW)], sem))
    copies.append(pltpu.async_copy(tab_hbm, tab_v, sem))
    for c in copies:
        c.wait()

    def chunk(c, carry):
        o = c * LANES
        q = q_v[pl.ds(o, LANES)]
        g0 = g_v[pl.ds(o, LANES)]
        g1 = g_v[pl.ds(RPW + o, LANES)]
        qt = q * NPAD
        s = []
        for k in range(4):
            r = r_v[pl.ds(k * RPW + o, LANES)]
            p = qt + r
            s0 = plsc.load_gather(tab_v, [p])
            s1 = plsc.load_gather(tab_v, [p + NPAD * NPAD])
            s.append(g0 * s0 + g1 * s1)
        inv = 1.0 / ((s[0] + s[1]) + (s[2] + s[3]))
        for k in range(4):
            out_v[pl.ds(k * RPW + o, LANES)] = s[k] * inv
        return carry

    lax.fori_loop(0, CHUNKS, chunk, 0)
    outs = [pltpu.async_copy(out_v.at[pl.ds(k * RPW, RPW)],
                             out_hbm.at[pl.ds(k * B + base, RPW)], sem)
            for k in range(4)]
    for c in outs:
        c.wait()


@functools.lru_cache(maxsize=1)
def _sc_rank():
    return pl.kernel(
        _sc_body,
        out_type=jax.ShapeDtypeStruct((4 * B,), jnp.float32),
        mesh=plsc.VectorSubcoreMesh(core_axis_name="c", subcore_axis_name="s"),
        compiler_params=pltpu.CompilerParams(needs_layout_passes=False),
        scratch_types=[
            pltpu.VMEM((RPW,), jnp.int32),
            pltpu.VMEM((4 * RPW,), jnp.int32),
            pltpu.VMEM((2 * RPW,), jnp.float32),
            pltpu.VMEM((2 * NPAD * NPAD,), jnp.float32),
            pltpu.VMEM((4 * RPW,), jnp.float32),
            pltpu.SemaphoreType.DMA,
        ],
    )


def kernel(given4rank1_stimulus_set, kernel_gate_weights, embed_table, w0, w1):
    tables = _make_tables(embed_table, w0, w1)
    flat = _sc_rank()(
        given4rank1_stimulus_set.astype(jnp.int32).T.reshape(-1),
        kernel_gate_weights.T.reshape(-1),
        tables,
    )
    return flat.reshape(4, B).T


# speedup vs baseline: 25.9204x; 1.0967x over previous
import functools

import jax
import jax.numpy as jnp
from jax import lax
from jax.experimental import pallas as pl
from jax.experimental.pallas import tpu as pltpu
from jax.experimental.pallas import tpu_sc as plsc

B = 16384
N_STIMULI = 30
N_DIM = 10
NPAD = 32
BETA = 10.0

NUM_CORES = 2
NUM_SUBCORES = 16
NW = NUM_CORES * NUM_SUBCORES
RPW = B // NW
LANES = 16
CHUNKS = RPW // LANES


def _table_body(ett_ref, w0_ref, w1_ref, out_ref):
    ett = jnp.concatenate(
        [ett_ref[...], jnp.zeros((N_DIM, 1), jnp.float32)], axis=1)
    et = ett.T
    d2_0 = jnp.zeros((NPAD, NPAD), jnp.float32)
    d2_1 = jnp.zeros((NPAD, NPAD), jnp.float32)
    for d in range(N_DIM):
        diff = et[:, d:d + 1] - ett[d:d + 1, :]
        sq = diff * diff
        d2_0 = d2_0 + w0_ref[0:1, d:d + 1] * sq
        d2_1 = d2_1 + w1_ref[0:1, d:d + 1] * sq
    for t, d2 in enumerate((d2_0, d2_1)):
        s = jnp.exp(-BETA * jnp.sqrt(d2))
        for qb in range(4):
            row = jnp.concatenate(
                [s[8 * qb + j: 8 * qb + j + 1, :] for j in range(8)], axis=1)
            out_ref[pl.ds(t * 1024 + qb * 256, 256)] = row.reshape(256)


def _make_tables(embed_table, w0, w1):
    return pl.pallas_call(
        _table_body,
        out_shape=jax.ShapeDtypeStruct((2 * NPAD * NPAD,), jnp.float32),
    )(embed_table.T, w0.reshape(1, N_DIM), w1.reshape(1, N_DIM))


def _sc_body(stim_hbm, gates_hbm, tab_hbm, out_hbm,
             q_v, r_v, g_v, tab_v, out_v, sem):
    wid = lax.axis_index("s") * NUM_CORES + lax.axis_index("c")
    base = wid * RPW
    copies = [pltpu.async_copy(stim_hbm.at[pl.ds(base, RPW)], q_v, sem)]
    for k in range(4):
        copies.append(pltpu.async_copy(
            stim_hbm.at[pl.ds((k + 1) * B + base, RPW)],
            r_v.at[pl.ds(k * RPW, RPW)], sem))
    copies.append(pltpu.async_copy(gates_hbm.at[pl.ds(2 * base, 2 * RPW)],
                                   g_v, sem))
    copies.append(pltpu.async_copy(tab_hbm, tab_v, sem))
    for c in copies:
        c.wait()

    for j in range(4):
        for cc in range(8):
            o = (j * 8 + cc) * LANES
            go = j * 256 + cc * LANES
            q = q_v[pl.ds(o, LANES)]
            g0 = g_v[pl.ds(go, LANES)]
            g1 = g_v[pl.ds(go + 128, LANES)]
            qt = q * NPAD
            s = []
            for k in range(4):
                r = r_v[pl.ds(k * RPW + o, LANES)]
                p = qt + r
                s0 = plsc.load_gather(tab_v, [p])
                s1 = plsc.load_gather(tab_v, [p + NPAD * NPAD])
                s.append(g0 * s0 + g1 * s1)
            inv = 1.0 / ((s[0] + s[1]) + (s[2] + s[3]))
            for k in range(4):
                out_v[pl.ds(j * 512 + k * 128 + cc * LANES, LANES)] = s[k] * inv

    pltpu.async_copy(out_v, out_hbm.at[pl.ds(4 * base, 4 * RPW)], sem).wait()


@functools.lru_cache(maxsize=1)
def _sc_rank():
    return pl.kernel(
        _sc_body,
        out_type=jax.ShapeDtypeStruct((4 * B,), jnp.float32),
        mesh=plsc.VectorSubcoreMesh(core_axis_name="c", subcore_axis_name="s"),
        compiler_params=pltpu.CompilerParams(needs_layout_passes=False),
        scratch_types=[
            pltpu.VMEM((RPW,), jnp.int32),
            pltpu.VMEM((4 * RPW,), jnp.int32),
            pltpu.VMEM((2 * RPW,), jnp.float32),
            pltpu.VMEM((2 * NPAD * NPAD,), jnp.float32),
            pltpu.VMEM((4 * RPW,), jnp.float32),
            pltpu.SemaphoreType.DMA,
        ],
    )


def kernel(given4rank1_stimulus_set, kernel_gate_weights, embed_table, w0, w1):
    tables = _make_tables(embed_table, w0, w1)
    gates_tiles = jnp.transpose(
        kernel_gate_weights.reshape(B // 128, 128, 2), (0, 2, 1)).reshape(-1)
    flat = _sc_rank()(
        given4rank1_stimulus_set.astype(jnp.int32).T.reshape(-1),
        gates_tiles,
        tables,
    )
    return jnp.transpose(flat.reshape(B // 128, 4, 128), (0, 2, 1)).reshape(B, 4)


# speedup vs baseline: 27.3726x; 1.0560x over previous
import functools

import jax
import jax.numpy as jnp
from jax import lax
from jax.experimental import pallas as pl
from jax.experimental.pallas import tpu as pltpu
from jax.experimental.pallas import tpu_sc as plsc

B = 16384
N_STIMULI = 30
N_DIM = 10
NPAD = 32
BETA = 10.0

NUM_CORES = 2
NUM_SUBCORES = 16
NW = NUM_CORES * NUM_SUBCORES
RPW = B // NW
LANES = 16
CHUNKS = RPW // LANES


def _table_body(ett_ref, w0_ref, w1_ref, out_ref):
    ett = jnp.concatenate(
        [ett_ref[...], jnp.zeros((N_DIM, 1), jnp.float32)], axis=1)
    et = ett.T
    d2_0 = jnp.zeros((NPAD, NPAD), jnp.float32)
    d2_1 = jnp.zeros((NPAD, NPAD), jnp.float32)
    for d in range(N_DIM):
        diff = et[:, d:d + 1] - ett[d:d + 1, :]
        sq = diff * diff
        d2_0 = d2_0 + w0_ref[0:1, d:d + 1] * sq
        d2_1 = d2_1 + w1_ref[0:1, d:d + 1] * sq
    for t, d2 in enumerate((d2_0, d2_1)):
        s = jnp.exp(-BETA * jnp.sqrt(d2))
        for qb in range(4):
            row = jnp.concatenate(
                [s[8 * qb + j: 8 * qb + j + 1, :] for j in range(8)], axis=1)
            out_ref[pl.ds(t * 1024 + qb * 256, 256)] = row.reshape(256)


def _make_tables(embed_table, w0, w1):
    return pl.pallas_call(
        _table_body,
        out_shape=jax.ShapeDtypeStruct((2 * NPAD * NPAD,), jnp.float32),
    )(embed_table.T, w0.reshape(1, N_DIM), w1.reshape(1, N_DIM))


def _sc_body(stim_hbm, gates_hbm, tab_hbm, out_hbm,
             sr_v, g_v, tab_v, out_v, sem):
    wid = lax.axis_index("s") * NUM_CORES + lax.axis_index("c")
    base = wid * RPW
    copies = [
        pltpu.async_copy(stim_hbm.at[:, pl.ds(base, RPW)], sr_v, sem),
        pltpu.async_copy(gates_hbm.at[pl.ds(2 * base, 2 * RPW)], g_v, sem),
        pltpu.async_copy(tab_hbm, tab_v, sem),
    ]
    for c in copies:
        c.wait()

    for j in range(4):
        for cc in range(8):
            o = (j * 8 + cc) * LANES
            go = j * 256 + cc * LANES
            q = sr_v[0, pl.ds(o, LANES)]
            g0 = g_v[pl.ds(go, LANES)]
            g1 = g_v[pl.ds(go + 128, LANES)]
            qt = q * NPAD
            s = []
            for k in range(4):
                r = sr_v[k + 1, pl.ds(o, LANES)]
                p = qt + r
                s0 = plsc.load_gather(tab_v, [p])
                s1 = plsc.load_gather(tab_v, [p + NPAD * NPAD])
                s.append(g0 * s0 + g1 * s1)
            inv = 1.0 / ((s[0] + s[1]) + (s[2] + s[3]))
            for k in range(4):
                out_v[pl.ds(j * 512 + k * 128 + cc * LANES, LANES)] = s[k] * inv

    pltpu.async_copy(out_v, out_hbm.at[pl.ds(4 * base, 4 * RPW)], sem).wait()


@functools.lru_cache(maxsize=1)
def _sc_rank():
    return pl.kernel(
        _sc_body,
        out_type=jax.ShapeDtypeStruct((4 * B,), jnp.float32),
        mesh=plsc.VectorSubcoreMesh(core_axis_name="c", subcore_axis_name="s"),
        compiler_params=pltpu.CompilerParams(needs_layout_passes=False),
        scratch_types=[
            pltpu.VMEM((5, RPW), jnp.int32),
            pltpu.VMEM((2 * RPW,), jnp.float32),
            pltpu.VMEM((2 * NPAD * NPAD,), jnp.float32),
            pltpu.VMEM((4 * RPW,), jnp.float32),
            pltpu.SemaphoreType.DMA,
        ],
    )


def kernel(given4rank1_stimulus_set, kernel_gate_weights, embed_table, w0, w1):
    tables = _make_tables(embed_table, w0, w1)
    gates_tiles = jnp.transpose(
        kernel_gate_weights.reshape(B // 128, 128, 2), (0, 2, 1)).reshape(-1)
    flat = _sc_rank()(
        given4rank1_stimulus_set.astype(jnp.int32).T,
        gates_tiles,
        tables,
    )
    return jnp.transpose(flat.reshape(B // 128, 4, 128), (0, 2, 1)).reshape(B, 4)
